# Initial kernel scaffold; baseline (speedup 1.0000x reference)
import functools

import jax
import jax.numpy as jnp
from jax import lax
from jax.experimental import pallas as pl
from jax.experimental.pallas import tpu as pltpu
from jax.experimental.pallas import tpu_sc as plsc

N = 32768
C = 64
NW = 32
ROWS = N // NW
L = 16
CG = C // L
U = 4

_mesh = plsc.VectorSubcoreMesh(core_axis_name="c", subcore_axis_name="s")


@functools.partial(
    pl.kernel,
    mesh=_mesh,
    out_type=jax.ShapeDtypeStruct((NW, C), jnp.float32),
    scratch_types=[
        pltpu.VMEM((ROWS, C), jnp.float32),
        pltpu.VMEM((C,), jnp.float32),
    ],
)
def _chunk_totals(x_hbm, tot_hbm, buf, tbuf):
    wid = lax.axis_index("s") * 2 + lax.axis_index("c")
    base = wid * ROWS
    pltpu.sync_copy(x_hbm.at[pl.ds(base, ROWS)], buf)

    def body(i, acc):
        out = list(acc)
        for u in range(U):
            r = i * U + u
            for g in range(CG):
                out[g] = out[g] + buf[r, pl.ds(g * L, L)]
        return tuple(out)

    acc = tuple(jnp.zeros((L,), jnp.float32) for _ in range(CG))
    acc = lax.fori_loop(0, ROWS // U, body, acc)
    for g in range(CG):
        tbuf[pl.ds(g * L, L)] = acc[g]
    pltpu.sync_copy(tbuf, tot_hbm.at[wid])


@functools.partial(
    pl.kernel,
    mesh=_mesh,
    out_type=jax.ShapeDtypeStruct((N, C), jnp.float32),
    scratch_types=[
        pltpu.VMEM((ROWS, C), jnp.float32),
        pltpu.VMEM((NW, C), jnp.float32),
    ],
)
def _chunk_scan(x_hbm, tot_hbm, out_hbm, buf, totv):
    wid = lax.axis_index("s") * 2 + lax.axis_index("c")
    base = wid * ROWS
    pltpu.sync_copy(x_hbm.at[pl.ds(base, ROWS)], buf)
    pltpu.sync_copy(tot_hbm, totv)

    def tbody(j, acc):
        m = (j < wid).astype(jnp.float32)
        return tuple(acc[g] + totv[j, pl.ds(g * L, L)] * m for g in range(CG))

    carry = lax.fori_loop(
        0, NW, tbody,
        tuple(jnp.full((L,), -1.0, jnp.float32) for _ in range(CG)))

    def body(i, c):
        c = list(c)
        for u in range(U):
            r = i * U + u
            for g in range(CG):
                c[g] = c[g] + buf[r, pl.ds(g * L, L)]
                buf[r, pl.ds(g * L, L)] = c[g]
        return tuple(c)

    lax.fori_loop(0, ROWS // U, body, carry)
    pltpu.sync_copy(buf, out_hbm.at[pl.ds(base, ROWS)])


def kernel(mask_i):
    tot = _chunk_totals(mask_i)
    return _chunk_scan(mask_i, tot)


# speedup vs baseline: 1.0019x; 1.0019x over previous
import functools

import jax
import jax.numpy as jnp
from jax import lax
from jax.experimental import pallas as pl
from jax.experimental.pallas import tpu as pltpu
from jax.experimental.pallas import tpu_sc as plsc

N = 32768
C = 64
NW = 32
ROWS = N // NW
SB = 256
NSB = ROWS // SB
L = 16
CG = C // L
U = 4

_mesh = plsc.VectorSubcoreMesh(core_axis_name="c", subcore_axis_name="s")


@functools.partial(
    pl.kernel,
    mesh=_mesh,
    out_type=jax.ShapeDtypeStruct((NW, C), jnp.float32),
    scratch_types=[
        pltpu.VMEM((SB, C), jnp.float32),
        pltpu.VMEM((C,), jnp.float32),
    ],
)
def _chunk_totals(x_hbm, tot_hbm, buf, tbuf):
    wid = lax.axis_index("s") * 2 + lax.axis_index("c")
    base = wid * ROWS

    def sub(s, acc):
        pltpu.sync_copy(x_hbm.at[pl.ds(base + s * SB, SB)], buf)

        def body(i, a):
            out = list(a)
            for u in range(U):
                r = i * U + u
                for g in range(CG):
                    out[g] = out[g] + buf[r, pl.ds(g * L, L)]
            return tuple(out)

        return lax.fori_loop(0, SB // U, body, acc)

    acc = tuple(jnp.zeros((L,), jnp.float32) for _ in range(CG))
    acc = lax.fori_loop(0, NSB, sub, acc)
    for g in range(CG):
        tbuf[pl.ds(g * L, L)] = acc[g]
    pltpu.sync_copy(tbuf, tot_hbm.at[wid])


@functools.partial(
    pl.kernel,
    mesh=_mesh,
    out_type=jax.ShapeDtypeStruct((N, C), jnp.float32),
    scratch_types=[
        pltpu.VMEM((SB, C), jnp.float32),
        pltpu.VMEM((NW, C), jnp.float32),
    ],
)
def _chunk_scan(x_hbm, tot_hbm, out_hbm, buf, totv):
    wid = lax.axis_index("s") * 2 + lax.axis_index("c")
    base = wid * ROWS
    pltpu.sync_copy(tot_hbm, totv)

    def tbody(j, acc):
        m = (j < wid).astype(jnp.float32)
        return tuple(acc[g] + totv[j, pl.ds(g * L, L)] * m for g in range(CG))

    carry = lax.fori_loop(
        0, NW, tbody,
        tuple(jnp.full((L,), -1.0, jnp.float32) for _ in range(CG)))

    def sub(s, carry):
        pltpu.sync_copy(x_hbm.at[pl.ds(base + s * SB, SB)], buf)

        def body(i, c):
            c = list(c)
            for u in range(U):
                r = i * U + u
                for g in range(CG):
                    c[g] = c[g] + buf[r, pl.ds(g * L, L)]
                    buf[r, pl.ds(g * L, L)] = c[g]
            return tuple(c)

        carry = lax.fori_loop(0, SB // U, body, carry)
        pltpu.sync_copy(buf, out_hbm.at[pl.ds(base + s * SB, SB)])
        return carry

    lax.fori_loop(0, NSB, sub, carry)


def kernel(mask_i):
    tot = _chunk_totals(mask_i)
    return _chunk_scan(mask_i, tot)
